# baseline (speedup 1.0000x reference)
import functools

import jax
import jax.numpy as jnp
from jax import lax
from jax.experimental import pallas as pl
from jax.experimental.pallas import tpu as pltpu
from jax.experimental.pallas import tpu_sc as plsc

_B = 16384
_F = 26
_VOCAB = 100000
_EMB = 64
_DENSE = 13

_NC, _NS = 2, 16
_NW = _NC * _NS
_ROWS_W = _B // _NW
_CHUNK = 64
_NCHUNK = _ROWS_W // _CHUNK
_IDX_PER_CHUNK = _CHUNK * _F
_GB = 128
_NGATHER = _IDX_PER_CHUNK // _GB
_IDX_ROWS = _IDX_PER_CHUNK // 128
_IDX_ROWS_PAD = 16
_NBLK = _B // _CHUNK


def _sc_embed_sum(sparse3d, offpat2d, table):
    mesh = plsc.VectorSubcoreMesh(core_axis_name="c", subcore_axis_name="s")

    @functools.partial(
        pl.kernel,
        out_type=jax.ShapeDtypeStruct((_B, _EMB), jnp.float32),
        mesh=mesh,
        scratch_types=[
            pltpu.VMEM((_IDX_ROWS_PAD, 128), jnp.int32),
            pltpu.VMEM((_IDX_ROWS_PAD, 128), jnp.int32),
            pltpu.VMEM((_IDX_PER_CHUNK, _EMB), jnp.float32),
            pltpu.VMEM((_CHUNK, _EMB), jnp.float32),
            pltpu.SemaphoreType.DMA,
        ],
        compiler_params=pltpu.CompilerParams(use_tc_tiling_on_sc=False),
    )
    def body(sparse_hbm, offpat_hbm, table_hbm, out_hbm,
             feat_v, off_v, rows_v, acc_v, sem):
        wid = lax.axis_index("s") * _NC + lax.axis_index("c")
        pltpu.sync_copy(offpat_hbm, off_v)

        def chunk_body(ci, carry):
            blk = wid * _NCHUNK + ci
            row0 = wid * _ROWS_W + ci * _CHUNK
            pltpu.sync_copy(sparse_hbm.at[blk], feat_v)
            for j in range(_IDX_ROWS):
                for c in range(128 // 16):
                    s = pl.ds(c * 16, 16)
                    feat_v[j, s] = feat_v[j, s] + off_v[j, s]
            copies = [
                pltpu.async_copy(table_hbm.at[feat_v.at[j]],
                                 rows_v.at[pl.ds(j * _GB, _GB)], sem)
                for j in range(_NGATHER)
            ]
            for cp in copies:
                cp.wait()

            def row_body(r, c2):
                p = r * _F
                for g in range(_EMB // 16):
                    s = pl.ds(g * 16, 16)
                    a = rows_v[p, s]
                    for f in range(1, _F):
                        a = a + rows_v[p + f, s]
                    acc_v[r, s] = a
                return c2

            lax.fori_loop(0, _CHUNK, row_body, 0, unroll=2)
            pltpu.sync_copy(acc_v, out_hbm.at[pl.ds(row0, _CHUNK)])
            return carry

        lax.fori_loop(0, _NCHUNK, chunk_body, 0)

    return body(sparse3d, offpat2d, table)


def _tc_combine(embed_sum, dense_p, lin_Wp, lin_b2, lin_bias2,
                w1, b1_2, w2, b2_2, w3_2, b3_2):
    bt = 2048
    grid = (_B // bt,)

    def body(emb_ref, den_ref, lw_ref, lb_ref, lbias_ref,
             w1_ref, b1_ref, w2_ref, b2_ref, w3_ref, b3_ref, out_ref):
        ex = emb_ref[...] * (1.0 / _F)
        srow = jnp.sum(ex, axis=1)
        ssq = jnp.sum(ex * ex, axis=1)
        fm = 0.5 * (srow * srow - ssq)
        lin = (jnp.dot(den_ref[...], lw_ref[...],
                       preferred_element_type=jnp.float32)
               + lb_ref[...] + lbias_ref[...])
        cat = jnp.concatenate([ex, lin], axis=1)
        h = jnp.maximum(jnp.dot(cat, w1_ref[...],
                                preferred_element_type=jnp.float32)
                        + b1_ref[...], 0.0)
        h = jnp.maximum(jnp.dot(h, w2_ref[...],
                                preferred_element_type=jnp.float32)
                        + b2_ref[...], 0.0)
        mlp = jnp.sum(h * w3_ref[...], axis=1) + b3_ref[0, 0]
        out_ref[...] = jax.nn.sigmoid(fm + mlp)

    full = lambda a: pl.BlockSpec(a.shape, lambda i: tuple(0 for _ in a.shape))
    return pl.pallas_call(
        body,
        grid=grid,
        in_specs=[
            pl.BlockSpec((bt, _EMB), lambda i: (i, 0)),
            pl.BlockSpec((bt, _EMB), lambda i: (i, 0)),
            full(lin_Wp), full(lin_b2), full(lin_bias2),
            full(w1), full(b1_2), full(w2), full(b2_2), full(w3_2),
            pl.BlockSpec((1, 1), lambda i: (0, 0), memory_space=pltpu.SMEM),
        ],
        out_specs=pl.BlockSpec((bt,), lambda i: (i,)),
        out_shape=jax.ShapeDtypeStruct((_B,), jnp.float32),
    )(embed_sum, dense_p, lin_Wp, lin_b2, lin_bias2,
      w1, b1_2, w2, b2_2, w3_2, b3_2)


def kernel(sparse_feats, dense_feats, table, lin_W, lin_b, lin_bias,
           w1, b1, w2, b2, w3, b3):
    sparse3d = jnp.pad(
        sparse_feats.astype(jnp.int32).reshape(_NBLK, _IDX_ROWS, 128),
        ((0, 0), (0, _IDX_ROWS_PAD - _IDX_ROWS), (0, 0)))
    offpat = jnp.pad(
        jnp.tile(jnp.arange(_F, dtype=jnp.int32) * _VOCAB,
                 _IDX_PER_CHUNK // _F).reshape(_IDX_ROWS, 128),
        ((0, _IDX_ROWS_PAD - _IDX_ROWS), (0, 0)))
    embed_sum = _sc_embed_sum(sparse3d, offpat, table)
    dense_p = jnp.pad(dense_feats, ((0, 0), (0, _EMB - _DENSE)))
    lin_Wp = jnp.pad(lin_W, ((0, _EMB - _DENSE), (0, 0)))
    return _tc_combine(
        embed_sum, dense_p, lin_Wp,
        lin_b.reshape(1, _EMB), lin_bias.reshape(1, _EMB),
        w1, b1.reshape(1, 128), w2, b2.reshape(1, _EMB),
        w3.reshape(1, _EMB), b3.reshape(1, 1))


# speedup vs baseline: 1.5679x; 1.5679x over previous
import functools

import jax
import jax.numpy as jnp
from jax import lax
from jax.experimental import pallas as pl
from jax.experimental.pallas import tpu as pltpu
from jax.experimental.pallas import tpu_sc as plsc

_B = 16384
_F = 26
_VOCAB = 100000
_EMB = 64
_DENSE = 13
_ROWS = _F * _VOCAB

_NC, _NS = 2, 16
_NW = _NC * _NS
_ROWS_W = _B // _NW
_CHUNK = 32
_NCHUNK = _ROWS_W // _CHUNK
_IDX_PER_CHUNK = _CHUNK * _F
_IDX_ROWS = 7
_IDX_ROWS_PAD = 8
_NBLK = _B // _CHUNK
_NG128 = _IDX_PER_CHUNK // 128
_GTAIL = _IDX_PER_CHUNK - _NG128 * 128

_TLANES = 4096
_TGRID = -(-_ROWS // _TLANES)
_PROWS = _TGRID * _TLANES


def _tc_transpose(table_t):
    def body(in_ref, out_ref):
        out_ref[:, 0:_EMB] = jnp.transpose(in_ref[...], (1, 0))

    return pl.pallas_call(
        body,
        grid=(_TGRID,),
        in_specs=[pl.BlockSpec((_EMB, _TLANES), lambda i: (0, i))],
        out_specs=pl.BlockSpec((_TLANES, 128), lambda i: (i, 0)),
        out_shape=jax.ShapeDtypeStruct((_PROWS, 128), jnp.float32),
    )(table_t)


def _sc_embed_sum(sparse3d, offpat2d, table128):
    mesh = plsc.VectorSubcoreMesh(core_axis_name="c", subcore_axis_name="s")

    @functools.partial(
        pl.kernel,
        out_type=jax.ShapeDtypeStruct((_B, _EMB), jnp.float32),
        mesh=mesh,
        scratch_types=[
            pltpu.VMEM((_IDX_ROWS_PAD, 128), jnp.int32),
            pltpu.VMEM((_IDX_ROWS_PAD, 128), jnp.int32),
            pltpu.VMEM((_IDX_PER_CHUNK, 128), jnp.float32),
            pltpu.VMEM((_CHUNK, _EMB), jnp.float32),
            pltpu.SemaphoreType.DMA,
        ],
    )
    def body(sparse_hbm, offpat_hbm, table_hbm, out_hbm,
             feat_v, off_v, rows_v, acc_v, sem):
        wid = lax.axis_index("s") * _NC + lax.axis_index("c")
        pltpu.sync_copy(offpat_hbm, off_v)

        def chunk_body(ci, carry):
            blk = wid * _NCHUNK + ci
            row0 = wid * _ROWS_W + ci * _CHUNK
            pltpu.sync_copy(sparse_hbm.at[blk], feat_v)
            for j in range(_IDX_ROWS):
                for c in range(128 // 16):
                    s = pl.ds(c * 16, 16)
                    feat_v[j, s] = feat_v[j, s] + off_v[j, s]
            copies = [
                pltpu.async_copy(table_hbm.at[feat_v.at[j]],
                                 rows_v.at[pl.ds(j * 128, 128)], sem)
                for j in range(_NG128)
            ]
            copies.append(
                pltpu.async_copy(
                    table_hbm.at[feat_v.at[_NG128, pl.ds(0, _GTAIL)]],
                    rows_v.at[pl.ds(_NG128 * 128, _GTAIL)], sem))
            for cp in copies:
                cp.wait()

            def row_body(r, c2):
                p = r * _F
                for g in range(_EMB // 16):
                    s = pl.ds(g * 16, 16)
                    a = rows_v[p, s]
                    for f in range(1, _F):
                        a = a + rows_v[p + f, s]
                    acc_v[r, s] = a
                return c2

            lax.fori_loop(0, _CHUNK, row_body, 0, unroll=2)
            pltpu.sync_copy(acc_v, out_hbm.at[pl.ds(row0, _CHUNK)])
            return carry

        lax.fori_loop(0, _NCHUNK, chunk_body, 0)

    return body(sparse3d, offpat2d, table128)


def _tc_combine(embed_sum, dense_p, lin_Wp, lin_b2, lin_bias2,
                w1, b1_2, w2, b2_2, w3_2, b3_2):
    bt = 2048
    grid = (_B // bt,)

    def body(emb_ref, den_ref, lw_ref, lb_ref, lbias_ref,
             w1_ref, b1_ref, w2_ref, b2_ref, w3_ref, b3_ref, out_ref):
        ex = emb_ref[...] * (1.0 / _F)
        srow = jnp.sum(ex, axis=1)
        ssq = jnp.sum(ex * ex, axis=1)
        fm = 0.5 * (srow * srow - ssq)
        lin = (jnp.dot(den_ref[...], lw_ref[...],
                       preferred_element_type=jnp.float32)
               + lb_ref[...] + lbias_ref[...])
        cat = jnp.concatenate([ex, lin], axis=1)
        h = jnp.maximum(jnp.dot(cat, w1_ref[...],
                                preferred_element_type=jnp.float32)
                        + b1_ref[...], 0.0)
        h = jnp.maximum(jnp.dot(h, w2_ref[...],
                                preferred_element_type=jnp.float32)
                        + b2_ref[...], 0.0)
        mlp = jnp.sum(h * w3_ref[...], axis=1) + b3_ref[0, 0]
        out_ref[...] = jax.nn.sigmoid(fm + mlp)

    full = lambda a: pl.BlockSpec(a.shape, lambda i: tuple(0 for _ in a.shape))
    return pl.pallas_call(
        body,
        grid=grid,
        in_specs=[
            pl.BlockSpec((bt, _EMB), lambda i: (i, 0)),
            pl.BlockSpec((bt, _EMB), lambda i: (i, 0)),
            full(lin_Wp), full(lin_b2), full(lin_bias2),
            full(w1), full(b1_2), full(w2), full(b2_2), full(w3_2),
            pl.BlockSpec((1, 1), lambda i: (0, 0), memory_space=pltpu.SMEM),
        ],
        out_specs=pl.BlockSpec((bt,), lambda i: (i,)),
        out_shape=jax.ShapeDtypeStruct((_B,), jnp.float32),
    )(embed_sum, dense_p, lin_Wp, lin_b2, lin_bias2,
      w1, b1_2, w2, b2_2, w3_2, b3_2)


def kernel(sparse_feats, dense_feats, table, lin_W, lin_b, lin_bias,
           w1, b1, w2, b2, w3, b3):
    sparse3d = jnp.pad(
        sparse_feats.astype(jnp.int32).reshape(_NBLK, _IDX_PER_CHUNK),
        ((0, 0), (0, _IDX_ROWS_PAD * 128 - _IDX_PER_CHUNK))
    ).reshape(_NBLK, _IDX_ROWS_PAD, 128)
    offpat = jnp.pad(
        jnp.tile(jnp.arange(_F, dtype=jnp.int32) * _VOCAB, _CHUNK),
        (0, _IDX_ROWS_PAD * 128 - _IDX_PER_CHUNK)
    ).reshape(_IDX_ROWS_PAD, 128)
    table128 = _tc_transpose(table.T)
    embed_sum = _sc_embed_sum(sparse3d, offpat, table128)
    dense_p = jnp.pad(dense_feats, ((0, 0), (0, _EMB - _DENSE)))
    lin_Wp = jnp.pad(lin_W, ((0, _EMB - _DENSE), (0, 0)))
    return _tc_combine(
        embed_sum, dense_p, lin_Wp,
        lin_b.reshape(1, _EMB), lin_bias.reshape(1, _EMB),
        w1, b1.reshape(1, 128), w2, b2.reshape(1, _EMB),
        w3.reshape(1, _EMB), b3.reshape(1, 1))


# speedup vs baseline: 1.8663x; 1.1903x over previous
import functools

import jax
import jax.numpy as jnp
from jax import lax
from jax.experimental import pallas as pl
from jax.experimental.pallas import tpu as pltpu
from jax.experimental.pallas import tpu_sc as plsc

_B = 16384
_F = 26
_VOCAB = 100000
_EMB = 64
_DENSE = 13
_ROWS = _F * _VOCAB

_NC, _NS = 2, 16
_NW = _NC * _NS
_ROWS_W = _B // _NW
_CHUNK = 32
_NCHUNK = _ROWS_W // _CHUNK
_IDX_PER_CHUNK = _CHUNK * _F
_IDX_ROWS = 7
_IDX_ROWS_PAD = 8
_NBLK = _B // _CHUNK
_NG128 = _IDX_PER_CHUNK // 128
_GTAIL = _IDX_PER_CHUNK - _NG128 * 128

_TLANES = 4096
_TGRID = 342
_PROWS = _TGRID * _TLANES
_BOFF = (_TGRID - 1) * _TLANES


def _tc_transpose(table_t):
    def body(ina_ref, inb_ref, out_ref):
        out_ref[:, 0:_EMB] = jnp.transpose(ina_ref[...], (1, 0))
        out_ref[:, _EMB:128] = jnp.transpose(inb_ref[...], (1, 0))

    return pl.pallas_call(
        body,
        grid=(_TGRID,),
        in_specs=[
            pl.BlockSpec((_EMB, _TLANES), lambda i: (0, i)),
            pl.BlockSpec(
                (_EMB, _TLANES),
                lambda i: (0, jnp.minimum(i + _TGRID - 1,
                                          _ROWS // _TLANES))),
        ],
        out_specs=pl.BlockSpec((_TLANES, 128), lambda i: (i, 0)),
        out_shape=jax.ShapeDtypeStruct((_PROWS, 128), jnp.float32),
    )(table_t, table_t)


def _sc_embed_sum(sparse3d, offpat2d, table128):
    mesh = plsc.VectorSubcoreMesh(core_axis_name="c", subcore_axis_name="s")

    @functools.partial(
        pl.kernel,
        out_type=jax.ShapeDtypeStruct((_B, _EMB), jnp.float32),
        mesh=mesh,
        scratch_types=[
            pltpu.VMEM((_IDX_ROWS_PAD, 128), jnp.int32),
            pltpu.VMEM((_IDX_ROWS_PAD, 128), jnp.int32),
            pltpu.VMEM((_IDX_PER_CHUNK, 128), jnp.float32),
            pltpu.VMEM((_CHUNK, _EMB), jnp.float32),
            pltpu.SemaphoreType.DMA,
        ],
    )
    def body(sparse_hbm, offpat_hbm, table_hbm, out_hbm,
             feat_v, off_v, rows_v, acc_v, sem):
        wid = lax.axis_index("s") * _NC + lax.axis_index("c")
        pltpu.sync_copy(offpat_hbm, off_v)

        def chunk_body(ci, carry):
            blk = wid * _NCHUNK + ci
            row0 = wid * _ROWS_W + ci * _CHUNK
            pltpu.sync_copy(sparse_hbm.at[blk], feat_v)
            for j in range(_IDX_ROWS):
                for c in range(128 // 16):
                    s = pl.ds(c * 16, 16)
                    feat_v[j, s] = feat_v[j, s] + off_v[j, s]
            copies = [
                pltpu.async_copy(table_hbm.at[feat_v.at[j]],
                                 rows_v.at[pl.ds(j * 128, 128)], sem)
                for j in range(_NG128)
            ]
            copies.append(
                pltpu.async_copy(
                    table_hbm.at[feat_v.at[_NG128, pl.ds(0, _GTAIL)]],
                    rows_v.at[pl.ds(_NG128 * 128, _GTAIL)], sem))
            for cp in copies:
                cp.wait()

            def row_body(r, c2):
                p = r * _F
                for g in range(_EMB // 16):
                    a = None
                    for f in range(_F):
                        half = 0 if f < 14 else _EMB
                        v = rows_v[p + f, pl.ds(half + g * 16, 16)]
                        a = v if a is None else a + v
                    acc_v[r, pl.ds(g * 16, 16)] = a
                return c2

            lax.fori_loop(0, _CHUNK, row_body, 0, unroll=2)
            pltpu.sync_copy(acc_v, out_hbm.at[pl.ds(row0, _CHUNK)])
            return carry

        lax.fori_loop(0, _NCHUNK, chunk_body, 0)

    return body(sparse3d, offpat2d, table128)


def _tc_combine(embed_sum, dense_p, lin_Wp, lin_b2, lin_bias2,
                w1, b1_2, w2, b2_2, w3_2, b3_2):
    bt = 2048
    grid = (_B // bt,)

    def body(emb_ref, den_ref, lw_ref, lb_ref, lbias_ref,
             w1_ref, b1_ref, w2_ref, b2_ref, w3_ref, b3_ref, out_ref):
        ex = emb_ref[...] * (1.0 / _F)
        srow = jnp.sum(ex, axis=1)
        ssq = jnp.sum(ex * ex, axis=1)
        fm = 0.5 * (srow * srow - ssq)
        lin = (jnp.dot(den_ref[...], lw_ref[...],
                       preferred_element_type=jnp.float32)
               + lb_ref[...] + lbias_ref[...])
        cat = jnp.concatenate([ex, lin], axis=1)
        h = jnp.maximum(jnp.dot(cat, w1_ref[...],
                                preferred_element_type=jnp.float32)
                        + b1_ref[...], 0.0)
        h = jnp.maximum(jnp.dot(h, w2_ref[...],
                                preferred_element_type=jnp.float32)
                        + b2_ref[...], 0.0)
        mlp = jnp.sum(h * w3_ref[...], axis=1) + b3_ref[0, 0]
        out_ref[...] = jax.nn.sigmoid(fm + mlp)

    full = lambda a: pl.BlockSpec(a.shape, lambda i: tuple(0 for _ in a.shape))
    return pl.pallas_call(
        body,
        grid=grid,
        in_specs=[
            pl.BlockSpec((bt, _EMB), lambda i: (i, 0)),
            pl.BlockSpec((bt, _EMB), lambda i: (i, 0)),
            full(lin_Wp), full(lin_b2), full(lin_bias2),
            full(w1), full(b1_2), full(w2), full(b2_2), full(w3_2),
            pl.BlockSpec((1, 1), lambda i: (0, 0), memory_space=pltpu.SMEM),
        ],
        out_specs=pl.BlockSpec((bt,), lambda i: (i,)),
        out_shape=jax.ShapeDtypeStruct((_B,), jnp.float32),
    )(embed_sum, dense_p, lin_Wp, lin_b2, lin_bias2,
      w1, b1_2, w2, b2_2, w3_2, b3_2)


def kernel(sparse_feats, dense_feats, table, lin_W, lin_b, lin_bias,
           w1, b1, w2, b2, w3, b3):
    sparse3d = jnp.pad(
        sparse_feats.astype(jnp.int32).reshape(_NBLK, _IDX_PER_CHUNK),
        ((0, 0), (0, _IDX_ROWS_PAD * 128 - _IDX_PER_CHUNK))
    ).reshape(_NBLK, _IDX_ROWS_PAD, 128)
    field_off = jnp.array(
        [f * _VOCAB - (_BOFF if f >= 14 else 0) for f in range(_F)],
        dtype=jnp.int32)
    offpat = jnp.pad(
        jnp.tile(field_off, _CHUNK),
        (0, _IDX_ROWS_PAD * 128 - _IDX_PER_CHUNK)
    ).reshape(_IDX_ROWS_PAD, 128)
    table128 = _tc_transpose(table.T)
    embed_sum = _sc_embed_sum(sparse3d, offpat, table128)
    dense_p = jnp.pad(dense_feats, ((0, 0), (0, _EMB - _DENSE)))
    lin_Wp = jnp.pad(lin_W, ((0, _EMB - _DENSE), (0, 0)))
    return _tc_combine(
        embed_sum, dense_p, lin_Wp,
        lin_b.reshape(1, _EMB), lin_bias.reshape(1, _EMB),
        w1, b1.reshape(1, 128), w2, b2.reshape(1, _EMB),
        w3.reshape(1, _EMB), b3.reshape(1, 1))


# speedup vs baseline: 1.9538x; 1.0469x over previous
import functools

import jax
import jax.numpy as jnp
from jax import lax
from jax.experimental import pallas as pl
from jax.experimental.pallas import tpu as pltpu
from jax.experimental.pallas import tpu_sc as plsc

_B = 16384
_F = 26
_VOCAB = 100000
_EMB = 64
_DENSE = 13
_ROWS = _F * _VOCAB

_NC, _NS = 2, 16
_NW = _NC * _NS
_ROWS_W = _B // _NW
_CHUNK = 32
_NCHUNK = _ROWS_W // _CHUNK
_IDX_PER_CHUNK = _CHUNK * _F
_IDX_ROWS = 7
_IDX_ROWS_PAD = 8
_NBLK = _B // _CHUNK
_NG128 = _IDX_PER_CHUNK // 128
_GTAIL = _IDX_PER_CHUNK - _NG128 * 128

_TLANES = 4096
_TGRID = 318
_PROWS = _TGRID * _TLANES
_BOFF = (_TGRID - 1) * _TLANES
_FSPLIT = 13


def _tc_transpose(table_t):
    def body(ina_ref, inb_ref, out_ref):
        out_ref[:, 0:_EMB] = jnp.transpose(ina_ref[...], (1, 0))
        out_ref[:, _EMB:128] = jnp.transpose(inb_ref[...], (1, 0))

    return pl.pallas_call(
        body,
        grid=(_TGRID,),
        in_specs=[
            pl.BlockSpec((_EMB, _TLANES), lambda i: (0, i)),
            pl.BlockSpec(
                (_EMB, _TLANES),
                lambda i: (0, jnp.minimum(i + _TGRID - 1,
                                          _ROWS // _TLANES))),
        ],
        out_specs=pl.BlockSpec((_TLANES, 128), lambda i: (i, 0)),
        out_shape=jax.ShapeDtypeStruct((_PROWS, 128), jnp.float32),
    )(table_t, table_t)


def _sc_embed_sum(sparse3d, offpat2d, table128):
    mesh = plsc.VectorSubcoreMesh(core_axis_name="c", subcore_axis_name="s")

    @functools.partial(
        pl.kernel,
        out_type=jax.ShapeDtypeStruct((_B, _EMB), jnp.float32),
        mesh=mesh,
        scratch_types=[
            pltpu.VMEM((_IDX_ROWS_PAD, 128), jnp.int32),
            pltpu.VMEM((_IDX_ROWS_PAD, 128), jnp.int32),
            pltpu.VMEM((_IDX_PER_CHUNK, 128), jnp.float32),
            pltpu.VMEM((_CHUNK, _EMB), jnp.float32),
            pltpu.SemaphoreType.DMA,
        ],
    )
    def body(sparse_hbm, offpat_hbm, table_hbm, out_hbm,
             feat_v, off_v, rows_v, acc_v, sem):
        wid = lax.axis_index("s") * _NC + lax.axis_index("c")
        pltpu.sync_copy(offpat_hbm, off_v)

        def chunk_body(ci, carry):
            blk = wid * _NCHUNK + ci
            row0 = wid * _ROWS_W + ci * _CHUNK
            pltpu.sync_copy(sparse_hbm.at[blk], feat_v)
            for j in range(_IDX_ROWS):
                for c in range(128 // 16):
                    s = pl.ds(c * 16, 16)
                    feat_v[j, s] = feat_v[j, s] + off_v[j, s]
            copies = [
                pltpu.async_copy(table_hbm.at[feat_v.at[j]],
                                 rows_v.at[pl.ds(j * 128, 128)], sem)
                for j in range(_NG128)
            ]
            copies.append(
                pltpu.async_copy(
                    table_hbm.at[feat_v.at[_NG128, pl.ds(0, _GTAIL)]],
                    rows_v.at[pl.ds(_NG128 * 128, _GTAIL)], sem))
            for cp in copies:
                cp.wait()

            def row_body(r, c2):
                p = r * _F
                for g in range(_EMB // 16):
                    a = None
                    for f in range(_F):
                        half = 0 if f < _FSPLIT else _EMB
                        v = rows_v[p + f, pl.ds(half + g * 16, 16)]
                        a = v if a is None else a + v
                    acc_v[r, pl.ds(g * 16, 16)] = a
                return c2

            lax.fori_loop(0, _CHUNK, row_body, 0, unroll=2)
            pltpu.sync_copy(acc_v, out_hbm.at[pl.ds(row0, _CHUNK)])
            return carry

        lax.fori_loop(0, _NCHUNK, chunk_body, 0)

    return body(sparse3d, offpat2d, table128)


def _tc_combine(embed_sum, dense_p, lin_Wp, lin_b2, lin_bias2,
                w1, b1_2, w2, b2_2, w3_2, b3_2):
    bt = 2048
    grid = (_B // bt,)

    def body(emb_ref, den_ref, lw_ref, lb_ref, lbias_ref,
             w1_ref, b1_ref, w2_ref, b2_ref, w3_ref, b3_ref, out_ref):
        ex = emb_ref[...] * (1.0 / _F)
        srow = jnp.sum(ex, axis=1)
        ssq = jnp.sum(ex * ex, axis=1)
        fm = 0.5 * (srow * srow - ssq)
        lin = (jnp.dot(den_ref[...], lw_ref[...],
                       preferred_element_type=jnp.float32)
               + lb_ref[...] + lbias_ref[...])
        cat = jnp.concatenate([ex, lin], axis=1)
        h = jnp.maximum(jnp.dot(cat, w1_ref[...],
                                preferred_element_type=jnp.float32)
                        + b1_ref[...], 0.0)
        h = jnp.maximum(jnp.dot(h, w2_ref[...],
                                preferred_element_type=jnp.float32)
                        + b2_ref[...], 0.0)
        mlp = jnp.sum(h * w3_ref[...], axis=1) + b3_ref[0, 0]
        out_ref[...] = jax.nn.sigmoid(fm + mlp)

    full = lambda a: pl.BlockSpec(a.shape, lambda i: tuple(0 for _ in a.shape))
    return pl.pallas_call(
        body,
        grid=grid,
        in_specs=[
            pl.BlockSpec((bt, _EMB), lambda i: (i, 0)),
            pl.BlockSpec((bt, _EMB), lambda i: (i, 0)),
            full(lin_Wp), full(lin_b2), full(lin_bias2),
            full(w1), full(b1_2), full(w2), full(b2_2), full(w3_2),
            pl.BlockSpec((1, 1), lambda i: (0, 0), memory_space=pltpu.SMEM),
        ],
        out_specs=pl.BlockSpec((bt,), lambda i: (i,)),
        out_shape=jax.ShapeDtypeStruct((_B,), jnp.float32),
    )(embed_sum, dense_p, lin_Wp, lin_b2, lin_bias2,
      w1, b1_2, w2, b2_2, w3_2, b3_2)


def kernel(sparse_feats, dense_feats, table, lin_W, lin_b, lin_bias,
           w1, b1, w2, b2, w3, b3):
    sparse3d = jnp.pad(
        sparse_feats.astype(jnp.int32).reshape(_NBLK, _IDX_PER_CHUNK),
        ((0, 0), (0, _IDX_ROWS_PAD * 128 - _IDX_PER_CHUNK))
    ).reshape(_NBLK, _IDX_ROWS_PAD, 128)
    field_off = jnp.array(
        [f * _VOCAB - (_BOFF if f >= _FSPLIT else 0) for f in range(_F)],
        dtype=jnp.int32)
    offpat = jnp.pad(
        jnp.tile(field_off, _CHUNK),
        (0, _IDX_ROWS_PAD * 128 - _IDX_PER_CHUNK)
    ).reshape(_IDX_ROWS_PAD, 128)
    table128 = _tc_transpose(table.T)
    embed_sum = _sc_embed_sum(sparse3d, offpat, table128)
    dense_p = jnp.pad(dense_feats, ((0, 0), (0, _EMB - _DENSE)))
    lin_Wp = jnp.pad(lin_W, ((0, _EMB - _DENSE), (0, 0)))
    return _tc_combine(
        embed_sum, dense_p, lin_Wp,
        lin_b.reshape(1, _EMB), lin_bias.reshape(1, _EMB),
        w1, b1.reshape(1, 128), w2, b2.reshape(1, _EMB),
        w3.reshape(1, _EMB), b3.reshape(1, 1))
